# baseline (speedup 1.0000x reference)
import functools

import jax
import jax.numpy as jnp
from jax import lax
from jax.experimental import pallas as pl
from jax.experimental.pallas import tpu as pltpu
from jax.experimental.pallas import tpu_sc as plsc

N = 50000
H = 64
HW = 32
L = 3

BLK = 512
NP = 50176
ACC_ROWS = 50432
TRASH = NP
EC = 128
ZROWS = ACC_ROWS // 16

E_FK_PAD = 802816
E_C_PAD = 57344

WO_FULL = 24
WO_TAIL = NP // 16 - WO_FULL * EC


@functools.lru_cache(maxsize=None)
def _make_layer_seg(with_counts):
    mesh = plsc.VectorSubcoreMesh(core_axis_name="c", subcore_axis_name="s")
    out_type = [jax.ShapeDtypeStruct((2, NP, HW), jnp.float32)] * 3
    scratch = [
        pltpu.VMEM_SHARED((ACC_ROWS, HW), jnp.float32),
        pltpu.VMEM((2, EC), jnp.int32),
        pltpu.VMEM((2, EC), jnp.int32),
        pltpu.VMEM((2, EC), jnp.int32),
        pltpu.VMEM((2, EC, HW), jnp.float32),
        pltpu.SemaphoreType.DMA((2,)),
        pltpu.SemaphoreType.DMA((2,)),
        pltpu.SemaphoreType.DMA((2,)),
    ]
    if with_counts:
        out_type = out_type + [jax.ShapeDtypeStruct((NP,), jnp.float32)] * 3
        scratch = scratch + [
            pltpu.VMEM_SHARED((ACC_ROWS,), jnp.float32),
            pltpu.VMEM((ZROWS,), jnp.float32),
            pltpu.VMEM((EC,), jnp.float32),
            pltpu.SemaphoreType.DMA((2,)),
        ]

    @functools.partial(
        pl.kernel,
        mesh=mesh,
        compiler_params=pltpu.CompilerParams(use_tc_tiling_on_sc=False),
        out_type=out_type,
        scratch_types=scratch,
    )
    def seg(src_c, dst_c, src_f, dst_f, src_r, dst_r, xt_hbm, xc_hbm, *rest):
        if with_counts:
            (out_c, out_f, out_r, cc_hbm, cf_hbm, cr_hbm,
             acc, sidx, draw, dix, rows, isem, gsem, ssem,
             cacc, cbuf, ones, csem) = rest
        else:
            (out_c, out_f, out_r,
             acc, sidx, draw, dix, rows, isem, gsem, ssem) = rest
        c = lax.axis_index("c")
        s = lax.axis_index("s")

        if with_counts:
            for k in range(EC // 16):
                ones[pl.ds(k * 16, 16)] = jnp.ones((16,), jnp.float32)

        def phase(e_pad, src_hbm, dst_hbm, x_hbm, out_hbm, cnt_hbm):
            n_chunks = e_pad // (16 * EC)

            def zrow(i, carry):
                for b in range(2):
                    for h4 in range(HW // 16):
                        rows[b, i, pl.ds(h4 * 16, 16)] = (
                            jnp.zeros((16,), jnp.float32))
                return carry
            lax.fori_loop(0, EC, zrow, 0)
            z0 = s * ZROWS
            for j in range(24):
                pltpu.async_copy(rows.at[0],
                                 acc.at[pl.ds(z0 + j * EC, EC), :], isem.at[0])
            pltpu.async_copy(rows.at[0, pl.ds(0, 80), :],
                             acc.at[pl.ds(z0 + 24 * EC, 80), :], isem.at[0])
            for j in range(24):
                pltpu.make_async_copy(
                    rows.at[0], acc.at[pl.ds(z0 + j * EC, EC), :],
                    isem.at[0]).wait()
            pltpu.make_async_copy(
                rows.at[0, pl.ds(0, 80), :],
                acc.at[pl.ds(z0 + 24 * EC, 80), :], isem.at[0]).wait()
            if cnt_hbm is not None:
                def zc(i, carry):
                    cbuf[pl.ds(i * 16, 16)] = jnp.zeros((16,), jnp.float32)
                    return carry
                lax.fori_loop(0, ZROWS // 16, zc, 0)
                pltpu.sync_copy(cbuf, cacc.at[pl.ds(z0, ZROWS)])
            plsc.subcore_barrier()

            tbase = s * (e_pad // 16)

            def idx_start(ci, b):
                base = tbase + ci * EC
                pltpu.async_copy(src_hbm.at[pl.ds(base, EC)], sidx.at[b],
                                 isem.at[b])
                pltpu.async_copy(dst_hbm.at[pl.ds(base, EC)], draw.at[b],
                                 isem.at[b])

            def idx_wait(b):
                pltpu.make_async_copy(src_hbm.at[pl.ds(0, EC)], sidx.at[b],
                                      isem.at[b]).wait()
                pltpu.make_async_copy(dst_hbm.at[pl.ds(0, EC)], draw.at[b],
                                      isem.at[b]).wait()

            def gather_start(b):
                pltpu.async_copy(x_hbm.at[c].at[sidx.at[b]], rows.at[b],
                                 gsem.at[b])

            def gather_wait(b):
                pltpu.make_async_copy(x_hbm.at[c].at[sidx.at[b]], rows.at[b],
                                      gsem.at[b]).wait()

            def scat_start(b):
                pltpu.async_copy(rows.at[b], acc.at[dix.at[b]], ssem.at[b],
                                 add=True)
                if cnt_hbm is not None:
                    pltpu.async_copy(ones, cacc.at[dix.at[b]], csem.at[b],
                                     add=True)

            def scat_wait(b):
                pltpu.make_async_copy(rows.at[b], acc.at[dix.at[b]],
                                      ssem.at[b]).wait()
                if cnt_hbm is not None:
                    pltpu.make_async_copy(ones, cacc.at[dix.at[b]],
                                          csem.at[b]).wait()

            def dix_compute(b):
                for k in range(EC // 16):
                    dix[b, pl.ds(k * 16, 16)] = draw[b, pl.ds(k * 16, 16)]

            for k in range(EC // 16):
                t16 = TRASH + jnp.arange(16, dtype=jnp.int32) + k * 16
                dix[0, pl.ds(k * 16, 16)] = t16
                dix[1, pl.ds(k * 16, 16)] = t16
            idx_start(0, 0)
            idx_start(1, 1)
            scat_start(0)
            scat_start(1)
            idx_wait(0)
            scat_wait(0)
            dix_compute(0)
            gather_start(0)

            def body(g, carry):
                for b in range(2):
                    gather_wait(b)
                    scat_start(b)
                    idx_start(2 * g + b + 2, b)
                    b1 = 1 - b
                    idx_wait(b1)
                    scat_wait(b1)
                    dix_compute(b1)
                    gather_start(b1)
                return carry
            lax.fori_loop(0, n_chunks // 2 - 1, body, 0)

            gather_wait(0)
            scat_start(0)
            idx_wait(1)
            scat_wait(1)
            dix_compute(1)
            gather_start(1)
            gather_wait(1)
            scat_start(1)
            scat_wait(0)
            scat_wait(1)
            plsc.subcore_barrier()

            def wo_chunk(j, nr):
                b = j & 1
                r0 = s * (NP // 16) + j * EC
                if j >= 2:
                    pr0 = s * (NP // 16) + (j - 2) * EC
                    pltpu.make_async_copy(
                        rows.at[b, pl.ds(0, EC), :],
                        out_hbm.at[c, pl.ds(pr0, EC), :], gsem.at[b]).wait()
                pltpu.sync_copy(acc.at[pl.ds(r0, nr), :],
                                rows.at[b, pl.ds(0, nr), :])
                pltpu.async_copy(rows.at[b, pl.ds(0, nr), :],
                                 out_hbm.at[c, pl.ds(r0, nr), :], gsem.at[b])
            for j in range(WO_FULL):
                wo_chunk(j, EC)
            wo_chunk(WO_FULL, WO_TAIL)
            rpen = s * (NP // 16) + (WO_FULL - 1) * EC
            pltpu.make_async_copy(rows.at[1, pl.ds(0, EC), :],
                                  out_hbm.at[c, pl.ds(rpen, EC), :],
                                  gsem.at[1]).wait()
            rlast = s * (NP // 16) + WO_FULL * EC
            pltpu.make_async_copy(rows.at[0, pl.ds(0, WO_TAIL), :],
                                  out_hbm.at[c, pl.ds(rlast, WO_TAIL), :],
                                  gsem.at[0]).wait()
            if cnt_hbm is not None:
                r0 = c * (NP // 2) + s * (NP // 32)
                pltpu.sync_copy(cacc.at[pl.ds(r0, NP // 32)],
                                cbuf.at[pl.ds(0, NP // 32)])
                pltpu.sync_copy(cbuf.at[pl.ds(0, NP // 32)],
                                cnt_hbm.at[pl.ds(r0, NP // 32)])
            plsc.subcore_barrier()

        phase(E_C_PAD, src_c, dst_c, xt_hbm, out_c,
              cc_hbm if with_counts else None)
        phase(E_FK_PAD, src_f, dst_f, xc_hbm, out_f,
              cf_hbm if with_counts else None)
        phase(E_C_PAD, src_r, dst_r, xc_hbm, out_r,
              cr_hbm if with_counts else None)

    return seg


def _cat(a3):
    return jnp.concatenate([a3[0], a3[1]], axis=1)


def _split(m):
    return jnp.stack([m[:, :HW], m[:, HW:]], axis=0)


def _enc_body(x_ref, w_ref, b_ref, o_ref):
    o_ref[...] = _split(x_ref[...] * w_ref[...] + b_ref[...])


def _encode(x, w, b):
    grid = (NP // BLK,)
    return pl.pallas_call(
        _enc_body,
        grid=grid,
        in_specs=[
            pl.BlockSpec((BLK, 1), lambda i: (i, 0)),
            pl.BlockSpec((1, H), lambda i: (0, 0)),
            pl.BlockSpec((1, H), lambda i: (0, 0)),
        ],
        out_specs=pl.BlockSpec((2, BLK, HW), lambda i: (0, i, 0)),
        out_shape=jax.ShapeDtypeStruct((2, NP, HW), jnp.float32),
    )(x, w, b)


def _layer_body(ht, hc, sc_, sf, sr, cc, cf, cr, wl, bl_, wr, oht, ohc):
    h_t = _cat(ht[...])
    h_c = _cat(hc[...])
    mc = _cat(sc_[...]) * (1.0 / jnp.maximum(cc[...], 1.0))
    mf = _cat(sf[...]) * (1.0 / jnp.maximum(cf[...], 1.0))
    mr = _cat(sr[...]) * (1.0 / jnp.maximum(cr[...], 1.0))
    wsum = wr[0] + wr[1]
    col = (jnp.dot(mc, wl[0], preferred_element_type=jnp.float32)
           + jnp.dot(mf, wl[1], preferred_element_type=jnp.float32)
           + jnp.dot(h_c, wsum, preferred_element_type=jnp.float32)
           + bl_[0] + bl_[1]) * 0.5
    tab = (jnp.dot(mr, wl[2], preferred_element_type=jnp.float32)
           + bl_[2]
           + jnp.dot(h_t, wr[2], preferred_element_type=jnp.float32))
    oht[...] = _split(jnp.maximum(h_t + col, 0.0))
    ohc[...] = _split(jnp.maximum(h_c + tab, 0.0))


def _layer(ht, hc, sc_, sf, sr, cc, cf, cr, wl, bl_, wr):
    grid = (NP // BLK,)
    mat = pl.BlockSpec((2, BLK, HW), lambda i: (0, i, 0))
    vec = pl.BlockSpec((BLK, 1), lambda i: (i, 0))
    return pl.pallas_call(
        _layer_body,
        grid=grid,
        in_specs=[mat, mat, mat, mat, mat, vec, vec, vec,
                  pl.BlockSpec((3, H, H), lambda i: (0, 0, 0)),
                  pl.BlockSpec((3, 1, H), lambda i: (0, 0, 0)),
                  pl.BlockSpec((3, H, H), lambda i: (0, 0, 0))],
        out_specs=[mat, mat],
        out_shape=[jax.ShapeDtypeStruct((2, NP, HW), jnp.float32),
                   jax.ShapeDtypeStruct((2, NP, HW), jnp.float32)],
    )(ht, hc, sc_, sf, sr, cc, cf, cr, wl, bl_, wr)


def _cls_body(ht, hc, wt, bt, wc, bc, ot, oc):
    ot[...] = jax.nn.sigmoid(
        jnp.dot(_cat(ht[...]), wt[...], preferred_element_type=jnp.float32)
        + bt[...])
    oc[...] = jax.nn.sigmoid(
        jnp.dot(_cat(hc[...]), wc[...], preferred_element_type=jnp.float32)
        + bc[...])


def _classify(ht, hc, wt, bt, wc, bc):
    grid = (NP // BLK,)
    mat = pl.BlockSpec((2, BLK, HW), lambda i: (0, i, 0))
    return pl.pallas_call(
        _cls_body,
        grid=grid,
        in_specs=[mat, mat,
                  pl.BlockSpec((H, 1), lambda i: (0, 0)),
                  pl.BlockSpec((1, 1), lambda i: (0, 0)),
                  pl.BlockSpec((H, 1), lambda i: (0, 0)),
                  pl.BlockSpec((1, 1), lambda i: (0, 0))],
        out_specs=[pl.BlockSpec((BLK, 1), lambda i: (i, 0)),
                   pl.BlockSpec((BLK, 1), lambda i: (i, 0))],
        out_shape=[jax.ShapeDtypeStruct((NP, 1), jnp.float32),
                   jax.ShapeDtypeStruct((NP, 1), jnp.float32)],
    )(ht, hc, wt, bt, wc, bc)


def _pad_edges(ei, e_pad):
    e = ei.shape[1]
    pad = e_pad - e
    ar = jnp.arange(pad, dtype=jnp.int32)
    src = jnp.concatenate([ei[0], ar % 8])
    dst = jnp.concatenate([ei[1], TRASH + (ar % 256)])
    return src, dst


def kernel(x_table, x_column, ei_contains, ei_foreign_key, ei_rev_contains,
           enc_t_W, enc_t_b, enc_c_W, enc_c_b, Wl, bl, Wr,
           cls_t_W, cls_t_b, cls_c_W, cls_c_b):
    xt = jnp.pad(x_table.astype(jnp.float32), (0, NP - N)).reshape(NP, 1)
    xc = jnp.pad(x_column.astype(jnp.float32), (0, NP - N)).reshape(NP, 1)
    h_t = _encode(xt, enc_t_W, enc_t_b.reshape(1, H))
    h_c = _encode(xc, enc_c_W, enc_c_b.reshape(1, H))

    src_c, dst_c = _pad_edges(ei_contains, E_C_PAD)
    src_f, dst_f = _pad_edges(ei_foreign_key, E_FK_PAD)
    src_r, dst_r = _pad_edges(ei_rev_contains, E_C_PAD)

    seg0 = _make_layer_seg(True)
    seg1 = _make_layer_seg(False)

    for l in range(L):
        if l == 0:
            s_c, s_f, s_r, cc, cf, cr = seg0(
                src_c, dst_c, src_f, dst_f, src_r, dst_r, h_t, h_c)
            cc = cc.reshape(NP, 1)
            cf = cf.reshape(NP, 1)
            cr = cr.reshape(NP, 1)
        else:
            s_c, s_f, s_r = seg1(
                src_c, dst_c, src_f, dst_f, src_r, dst_r, h_t, h_c)
        h_t, h_c = _layer(h_t, h_c, s_c, s_f, s_r, cc, cf, cr,
                          Wl[l], bl[l].reshape(3, 1, H), Wr[l])

    t_out, c_out = _classify(h_t, h_c, cls_t_W, cls_t_b.reshape(1, 1),
                             cls_c_W, cls_c_b.reshape(1, 1))
    return t_out.reshape(NP)[:N], c_out.reshape(NP)[:N]


# speedup vs baseline: 1.5967x; 1.5967x over previous
import functools

import jax
import jax.numpy as jnp
from jax import lax
from jax.experimental import pallas as pl
from jax.experimental.pallas import tpu as pltpu
from jax.experimental.pallas import tpu_sc as plsc

N = 50000
H = 64
HW = 32
L = 3

BLK = 512
NP = 50176
ACC_ROWS = 50240
CACC_ROWS = 50432
TRASH = NP
EC = 128
NB = 4
ZROWS = ACC_ROWS // 16
CZROWS = CACC_ROWS // 16

E_FK_PAD = 802816
E_C_PAD = 57344

WO_FULL = 24
WO_TAIL = NP // 16 - WO_FULL * EC


@functools.lru_cache(maxsize=None)
def _make_seg(with_counts, which):
    nout = 2 if which == "cf" else 1
    mesh = plsc.VectorSubcoreMesh(core_axis_name="c", subcore_axis_name="s")
    out_type = [jax.ShapeDtypeStruct((2, NP, HW), jnp.float32)] * nout
    scratch = [
        pltpu.VMEM_SHARED((ACC_ROWS, HW), jnp.float32),
        pltpu.VMEM((NB, 2, EC), jnp.int32),
        pltpu.VMEM((NB, EC), jnp.int32),
        pltpu.VMEM((NB, EC, HW), jnp.float32),
        pltpu.SemaphoreType.DMA((NB,)),
        pltpu.SemaphoreType.DMA((NB,)),
        pltpu.SemaphoreType.DMA((NB,)),
    ]
    if with_counts:
        out_type = out_type + [jax.ShapeDtypeStruct((NP,), jnp.float32)] * nout
        scratch = scratch + [
            pltpu.VMEM_SHARED((CACC_ROWS,), jnp.float32),
            pltpu.VMEM((1600,), jnp.float32),
            pltpu.VMEM((EC,), jnp.float32),
            pltpu.SemaphoreType.DMA((NB,)),
        ]

    @functools.partial(
        pl.kernel,
        mesh=mesh,
        compiler_params=pltpu.CompilerParams(use_tc_tiling_on_sc=False),
        out_type=out_type,
        scratch_types=scratch,
    )
    def seg(*args):
        nin = 4 if which == "cf" else 2
        ins, rest = args[:nin], args[nin:]
        outs, rest = rest[:nout], rest[nout:]
        if with_counts:
            couts, rest = rest[:nout], rest[nout:]
            (acc, ibuf, dix, rows, isem, gsem, ssem,
             cacc, cbuf, ones, csem) = rest
        else:
            couts = (None,) * nout
            (acc, ibuf, dix, rows, isem, gsem, ssem) = rest
        c = lax.axis_index("c")
        s = lax.axis_index("s")

        if with_counts:
            for k in range(EC // 16):
                ones[pl.ds(k * 16, 16)] = jnp.ones((16,), jnp.float32)

        def phase(e_pad, ei_hbm, x_hbm, out_hbm, cnt_hbm):
            n_chunks = e_pad // (16 * EC)
            n4 = n_chunks // NB

            def zrow(i, carry):
                for b in range(NB):
                    for h4 in range(HW // 16):
                        rows[b, i, pl.ds(h4 * 16, 16)] = (
                            jnp.zeros((16,), jnp.float32))
                return carry
            lax.fori_loop(0, EC, zrow, 0)
            z0 = s * ZROWS
            for j in range(24):
                pltpu.async_copy(rows.at[0],
                                 acc.at[pl.ds(z0 + j * EC, EC), :], isem.at[0])
            pltpu.async_copy(rows.at[0, pl.ds(0, 68), :],
                             acc.at[pl.ds(z0 + 24 * EC, 68), :], isem.at[0])
            for j in range(24):
                pltpu.make_async_copy(
                    rows.at[0], acc.at[pl.ds(z0 + j * EC, EC), :],
                    isem.at[0]).wait()
            pltpu.make_async_copy(
                rows.at[0, pl.ds(0, 68), :],
                acc.at[pl.ds(z0 + 24 * EC, 68), :], isem.at[0]).wait()
            if cnt_hbm is not None:
                def zc(i, carry):
                    cbuf[pl.ds(i * 16, 16)] = jnp.zeros((16,), jnp.float32)
                    return carry
                lax.fori_loop(0, 100, zc, 0)
                cz0 = s * CZROWS
                pltpu.sync_copy(cbuf, cacc.at[pl.ds(cz0, 1600)])
                pltpu.sync_copy(cbuf.at[pl.ds(0, 1552)],
                                cacc.at[pl.ds(cz0 + 1600, 1552)])
            plsc.subcore_barrier()

            tbase = s * (e_pad // 16)

            def idx_start(ci, b):
                base = tbase + ci * EC
                pltpu.async_copy(ei_hbm.at[:, pl.ds(base, EC)], ibuf.at[b],
                                 isem.at[b])

            def idx_wait(b):
                pltpu.make_async_copy(ei_hbm.at[:, pl.ds(0, EC)], ibuf.at[b],
                                      isem.at[b]).wait()

            def gather_start(b):
                pltpu.async_copy(x_hbm.at[c].at[ibuf.at[b, 0]], rows.at[b],
                                 gsem.at[b])

            def gather_wait(b):
                pltpu.make_async_copy(x_hbm.at[c].at[ibuf.at[b, 0]],
                                      rows.at[b], gsem.at[b]).wait()

            def scat_start(b):
                pltpu.async_copy(rows.at[b], acc.at[dix.at[b]], ssem.at[b],
                                 add=True)
                if cnt_hbm is not None:
                    pltpu.async_copy(ones, cacc.at[dix.at[b]], csem.at[b],
                                     add=True)

            def scat_wait(b):
                pltpu.make_async_copy(rows.at[b], acc.at[dix.at[b]],
                                      ssem.at[b]).wait()
                if cnt_hbm is not None:
                    pltpu.make_async_copy(ones, cacc.at[dix.at[b]],
                                          csem.at[b]).wait()

            def dix_compute(b):
                for k in range(EC // 16):
                    dix[b, pl.ds(k * 16, 16)] = ibuf[b, 1, pl.ds(k * 16, 16)]

            def issue(b):
                idx_wait(b)
                scat_wait(b)
                dix_compute(b)
                gather_start(b)

            def complete(b2):
                gather_wait(b2)
                scat_start(b2)

            for k in range(EC // 16):
                t16 = TRASH + ((jnp.arange(16, dtype=jnp.int32) + k * 16)
                               & 63)
                for b in range(NB):
                    dix[b, pl.ds(k * 16, 16)] = t16
            for b in range(NB):
                idx_start(b, b)
                scat_start(b)
            for k in range(NB):
                issue(k)
                if k >= 2:
                    b2 = (k + 2) & 3
                    complete(b2)
                    idx_start(k + 2, b2)

            def body(g, carry):
                for k in range(NB):
                    issue(k)
                    b2 = (k + 2) & 3
                    complete(b2)
                    idx_start(NB * g + k + 2, b2)
                return carry
            lax.fori_loop(1, n4 - 1, body, 0)

            for k in range(NB):
                issue(k)
                b2 = (k + 2) & 3
                complete(b2)
                if k < 2:
                    idx_start(n_chunks - NB + k + 2, b2)
            complete(2)
            complete(3)
            for b in range(NB):
                scat_wait(b)
            plsc.subcore_barrier()

            def wo_chunk(j, nr):
                b = j & 1
                r0 = s * (NP // 16) + j * EC
                if j >= 2:
                    pr0 = s * (NP // 16) + (j - 2) * EC
                    pltpu.make_async_copy(
                        rows.at[b, pl.ds(0, EC), :],
                        out_hbm.at[c, pl.ds(pr0, EC), :], gsem.at[b]).wait()
                pltpu.sync_copy(acc.at[pl.ds(r0, nr), :],
                                rows.at[b, pl.ds(0, nr), :])
                pltpu.async_copy(rows.at[b, pl.ds(0, nr), :],
                                 out_hbm.at[c, pl.ds(r0, nr), :], gsem.at[b])
            for j in range(WO_FULL):
                wo_chunk(j, EC)
            wo_chunk(WO_FULL, WO_TAIL)
            rpen = s * (NP // 16) + (WO_FULL - 1) * EC
            pltpu.make_async_copy(rows.at[1, pl.ds(0, EC), :],
                                  out_hbm.at[c, pl.ds(rpen, EC), :],
                                  gsem.at[1]).wait()
            rlast = s * (NP // 16) + WO_FULL * EC
            pltpu.make_async_copy(rows.at[0, pl.ds(0, WO_TAIL), :],
                                  out_hbm.at[c, pl.ds(rlast, WO_TAIL), :],
                                  gsem.at[0]).wait()
            if cnt_hbm is not None:
                r0 = c * (NP // 2) + s * (NP // 32)
                pltpu.sync_copy(cacc.at[pl.ds(r0, NP // 32)],
                                cbuf.at[pl.ds(0, NP // 32)])
                pltpu.sync_copy(cbuf.at[pl.ds(0, NP // 32)],
                                cnt_hbm.at[pl.ds(r0, NP // 32)])
            plsc.subcore_barrier()

        if which == "cf":
            ei_c, ei_f, xt_hbm, xc_hbm = ins
            phase(E_C_PAD, ei_c, xt_hbm, outs[0], couts[0])
            phase(E_FK_PAD, ei_f, xc_hbm, outs[1], couts[1])
        else:
            ei_r, xc_hbm = ins
            phase(E_C_PAD, ei_r, xc_hbm, outs[0], couts[0])

    return seg


NP4 = NP // 4
BLK4 = BLK // 4


def _qgroup(a3, q):
    return jnp.concatenate([a3[0, :, HW * q:HW * q + HW],
                            a3[1, :, HW * q:HW * q + HW]], axis=1)


def _qpack(rs):
    return jnp.stack(
        [jnp.concatenate([r[:, :HW] for r in rs], axis=1),
         jnp.concatenate([r[:, HW:] for r in rs], axis=1)], axis=0)


def _enc_body(x_ref, w_ref, b_ref, o_ref):
    rs = []
    for q in range(4):
        rs.append(x_ref[q] * w_ref[...] + b_ref[...])
    o_ref[...] = _qpack(rs)


def _encode(x, w, b):
    grid = (NP4 // BLK4,)
    return pl.pallas_call(
        _enc_body,
        grid=grid,
        in_specs=[
            pl.BlockSpec((4, BLK4, 1), lambda i: (0, i, 0)),
            pl.BlockSpec((1, H), lambda i: (0, 0)),
            pl.BlockSpec((1, H), lambda i: (0, 0)),
        ],
        out_specs=pl.BlockSpec((2, BLK4, 128), lambda i: (0, i, 0)),
        out_shape=jax.ShapeDtypeStruct((2, NP4, 128), jnp.float32),
    )(x, w, b)


def _layer_t_body(ht, hc, sc_, sf, cc, cf, wl, bl_, wr, oht):
    wsum = wr[0] + wr[1]
    rs = []
    for q in range(4):
        h_t = _qgroup(ht[...], q)
        h_c = _qgroup(hc[...], q)
        mc = _qgroup(sc_[...], q) * (1.0 / jnp.maximum(cc[q], 1.0))
        mf = _qgroup(sf[...], q) * (1.0 / jnp.maximum(cf[q], 1.0))
        col = (jnp.dot(mc, wl[0], preferred_element_type=jnp.float32)
               + jnp.dot(mf, wl[1], preferred_element_type=jnp.float32)
               + jnp.dot(h_c, wsum, preferred_element_type=jnp.float32)
               + bl_[0] + bl_[1]) * 0.5
        rs.append(jnp.maximum(h_t + col, 0.0))
    oht[...] = _qpack(rs)


def _layer_c_body(ht, hc, sr, cr, wl, bl_, wr, ohc):
    rs = []
    for q in range(4):
        h_t = _qgroup(ht[...], q)
        h_c = _qgroup(hc[...], q)
        mr = _qgroup(sr[...], q) * (1.0 / jnp.maximum(cr[q], 1.0))
        tab = (jnp.dot(mr, wl[2], preferred_element_type=jnp.float32)
               + bl_[2]
               + jnp.dot(h_t, wr[2], preferred_element_type=jnp.float32))
        rs.append(jnp.maximum(h_c + tab, 0.0))
    ohc[...] = _qpack(rs)


def _layer_t(ht, hc, sc_, sf, cc, cf, wl, bl_, wr):
    grid = (NP4 // BLK4,)
    mat = pl.BlockSpec((2, BLK4, 128), lambda i: (0, i, 0))
    vec = pl.BlockSpec((4, BLK4, 1), lambda i: (0, i, 0))
    return pl.pallas_call(
        _layer_t_body,
        grid=grid,
        in_specs=[mat, mat, mat, mat, vec, vec,
                  pl.BlockSpec((3, H, H), lambda i: (0, 0, 0)),
                  pl.BlockSpec((3, 1, H), lambda i: (0, 0, 0)),
                  pl.BlockSpec((3, H, H), lambda i: (0, 0, 0))],
        out_specs=mat,
        out_shape=jax.ShapeDtypeStruct((2, NP4, 128), jnp.float32),
    )(ht, hc, sc_, sf, cc, cf, wl, bl_, wr)


def _layer_c(ht, hc, sr, cr, wl, bl_, wr):
    grid = (NP4 // BLK4,)
    mat = pl.BlockSpec((2, BLK4, 128), lambda i: (0, i, 0))
    vec = pl.BlockSpec((4, BLK4, 1), lambda i: (0, i, 0))
    return pl.pallas_call(
        _layer_c_body,
        grid=grid,
        in_specs=[mat, mat, mat, vec,
                  pl.BlockSpec((3, H, H), lambda i: (0, 0, 0)),
                  pl.BlockSpec((3, 1, H), lambda i: (0, 0, 0)),
                  pl.BlockSpec((3, H, H), lambda i: (0, 0, 0))],
        out_specs=mat,
        out_shape=jax.ShapeDtypeStruct((2, NP4, 128), jnp.float32),
    )(ht, hc, sr, cr, wl, bl_, wr)


def _cls_body(ht, hc, wt, bt, wc, bc, ot, oc):
    ts, cs = [], []
    for q in range(4):
        h_t = _qgroup(ht[...], q)
        h_c = _qgroup(hc[...], q)
        ts.append(jax.nn.sigmoid(
            jnp.dot(h_t, wt[...], preferred_element_type=jnp.float32)
            + bt[...]))
        cs.append(jax.nn.sigmoid(
            jnp.dot(h_c, wc[...], preferred_element_type=jnp.float32)
            + bc[...]))
    ot[...] = jnp.stack(ts, axis=0)
    oc[...] = jnp.stack(cs, axis=0)


def _classify(ht, hc, wt, bt, wc, bc):
    grid = (NP4 // BLK4,)
    mat = pl.BlockSpec((2, BLK4, 128), lambda i: (0, i, 0))
    vec = pl.BlockSpec((4, BLK4, 1), lambda i: (0, i, 0))
    return pl.pallas_call(
        _cls_body,
        grid=grid,
        in_specs=[mat, mat,
                  pl.BlockSpec((H, 1), lambda i: (0, 0)),
                  pl.BlockSpec((1, 1), lambda i: (0, 0)),
                  pl.BlockSpec((H, 1), lambda i: (0, 0)),
                  pl.BlockSpec((1, 1), lambda i: (0, 0))],
        out_specs=[vec, vec],
        out_shape=[jax.ShapeDtypeStruct((4, NP4, 1), jnp.float32),
                   jax.ShapeDtypeStruct((4, NP4, 1), jnp.float32)],
    )(ht, hc, wt, bt, wc, bc)


def _pad_edges(ei, e_pad):
    e = ei.shape[1]
    pad = e_pad - e
    ar = jnp.arange(pad, dtype=jnp.int32)
    src = jnp.concatenate([ei[0], ar % 8])
    dst = jnp.concatenate([ei[1], TRASH + (ar % 64)])
    return jnp.stack([src, dst])


def kernel(x_table, x_column, ei_contains, ei_foreign_key, ei_rev_contains,
           enc_t_W, enc_t_b, enc_c_W, enc_c_b, Wl, bl, Wr,
           cls_t_W, cls_t_b, cls_c_W, cls_c_b):
    def interleave(v):
        return v.reshape(NP4, 4).transpose(1, 0).reshape(4, NP4, 1)

    xt = interleave(jnp.pad(x_table.astype(jnp.float32), (0, NP - N)))
    xc = interleave(jnp.pad(x_column.astype(jnp.float32), (0, NP - N)))
    h_t = _encode(xt, enc_t_W, enc_t_b.reshape(1, H))
    h_c = _encode(xc, enc_c_W, enc_c_b.reshape(1, H))

    eic = _pad_edges(ei_contains, E_C_PAD)
    eif = _pad_edges(ei_foreign_key, E_FK_PAD)
    eir = _pad_edges(ei_rev_contains, E_C_PAD)

    def sc_view(a):
        return a.reshape(2, NP, HW)

    def tc_view(a):
        return a.reshape(2, NP4, 128)

    for l in range(L):
        ht_sc = sc_view(h_t)
        hc_sc = sc_view(h_c)
        if l == 0:
            s_r, cr = _make_seg(True, "r")(eir, hc_sc)
            s_c, s_f, cc, cf = _make_seg(True, "cf")(eic, eif, ht_sc, hc_sc)
            cc = interleave(cc)
            cf = interleave(cf)
            cr = interleave(cr)
        else:
            s_r, = _make_seg(False, "r")(eir, hc_sc)
            s_c, s_f = _make_seg(False, "cf")(eic, eif, ht_sc, hc_sc)
        bl3 = bl[l].reshape(3, 1, H)
        h_c_new = _layer_c(h_t, h_c, tc_view(s_r), cr, Wl[l], bl3, Wr[l])
        h_t = _layer_t(h_t, h_c, tc_view(s_c), tc_view(s_f), cc, cf,
                       Wl[l], bl3, Wr[l])
        h_c = h_c_new

    t_out, c_out = _classify(h_t, h_c, cls_t_W, cls_t_b.reshape(1, 1),
                             cls_c_W, cls_c_b.reshape(1, 1))
    t_full = t_out.reshape(4, NP4).transpose(1, 0).reshape(NP)
    c_full = c_out.reshape(4, NP4).transpose(1, 0).reshape(NP)
    return t_full[:N], c_full[:N]
